# Initial kernel scaffold; baseline (speedup 1.0000x reference)
import jax
import jax.numpy as jnp
from jax.experimental import pallas as pl


def kernel(x, emb, W1, b1, W2, b2):
    raise NotImplementedError("write your pallas kernel here")


# speedup vs baseline: 12.3090x; 12.3090x over previous
import functools

import jax
import jax.numpy as jnp
from jax import lax
from jax.experimental import pallas as pl
from jax.experimental.pallas import tpu as pltpu
from jax.experimental.pallas import tpu_sc as plsc

B, L = 1024, 200
N = B * L
D = 128
N_TAGS = 45

_info = plsc.get_sparse_core_info()
NC, NS = _info.num_cores, _info.num_subcores
NW = NC * NS
PER_W = N // NW
CH = 128
N_CH = PER_W // CH


def _sc_gather_kernel(idx_hbm, table_hbm, out_hbm, idx_v, buf, sem):
    wid = lax.axis_index("s") * NC + lax.axis_index("c")
    base = wid * PER_W
    pltpu.sync_copy(idx_hbm.at[pl.ds(base, PER_W)], idx_v)

    def chunk(i, carry):
        off = i * CH
        pltpu.async_copy(table_hbm.at[idx_v.at[pl.ds(off, CH)]], buf, sem).wait()
        pltpu.sync_copy(buf, out_hbm.at[pl.ds(base + off, CH)])
        return carry

    lax.fori_loop(0, N_CH, chunk, 0, unroll=False)


def _sc_gather(idx, table):
    mesh = plsc.VectorSubcoreMesh(core_axis_name="c", subcore_axis_name="s")
    k = functools.partial(
        pl.kernel,
        mesh=mesh,
        out_type=jax.ShapeDtypeStruct((N, D), jnp.float32),
        scratch_types=[
            pltpu.VMEM((PER_W,), jnp.int32),
            pltpu.VMEM((CH, D), jnp.float32),
            pltpu.SemaphoreType.DMA,
        ],
    )(_sc_gather_kernel)
    return k(idx, table)


BLK = 2048


def _mlp_kernel(g_ref, w1t_ref, b1_ref, w2t_ref, b2_ref, out_ref):
    h = jnp.maximum(
        jnp.dot(g_ref[...], w1t_ref[...], preferred_element_type=jnp.float32)
        + b1_ref[...],
        0.0,
    )
    out_ref[...] = (
        jnp.dot(h, w2t_ref[...], preferred_element_type=jnp.float32) + b2_ref[...]
    )


def _tc_mlp(g, w1t, b1, w2t, b2):
    grid = (N // BLK,)
    return pl.pallas_call(
        _mlp_kernel,
        grid=grid,
        in_specs=[
            pl.BlockSpec((BLK, D), lambda i: (i, 0)),
            pl.BlockSpec((D, D), lambda i: (0, 0)),
            pl.BlockSpec((1, D), lambda i: (0, 0)),
            pl.BlockSpec((D, N_TAGS), lambda i: (0, 0)),
            pl.BlockSpec((1, N_TAGS), lambda i: (0, 0)),
        ],
        out_specs=pl.BlockSpec((BLK, N_TAGS), lambda i: (i, 0)),
        out_shape=jax.ShapeDtypeStruct((N, N_TAGS), jnp.float32),
    )(g, w1t, b1, w2t, b2)


def kernel(x, emb, W1, b1, W2, b2):
    idx = x.reshape(-1).astype(jnp.int32)
    g = _sc_gather(idx, emb)
    out = _tc_mlp(g, W1.T, b1.reshape(1, D), W2.T, b2.reshape(1, N_TAGS))
    return out.reshape(B, L, N_TAGS)


# speedup vs baseline: 15.3050x; 1.2434x over previous
import functools

import jax
import jax.numpy as jnp
from jax import lax
from jax.experimental import pallas as pl
from jax.experimental.pallas import tpu as pltpu
from jax.experimental.pallas import tpu_sc as plsc

B, L = 1024, 200
N = B * L
D = 128
N_TAGS = 45

_info = plsc.get_sparse_core_info()
NC, NS = _info.num_cores, _info.num_subcores
NW = NC * NS
PER_W = N // NW
CH = 128
N_CH = PER_W // CH


def _sc_gather_kernel(idx_hbm, table_hbm, out_hbm, idx_v, buf0, buf1, sem0, sem1):
    wid = lax.axis_index("s") * NC + lax.axis_index("c")
    base = wid * PER_W
    pltpu.sync_copy(idx_hbm.at[pl.ds(base, PER_W)], idx_v)

    pltpu.async_copy(table_hbm.at[idx_v.at[pl.ds(0, CH)]], buf0, sem0)
    pltpu.async_copy(table_hbm.at[idx_v.at[pl.ds(CH, CH)]], buf1, sem1)

    def pair(i, carry):
        for b, buf, sem in ((0, buf0, sem0), (1, buf1, sem1)):
            j = 2 * i + b
            off = j * CH
            pltpu.make_async_copy(
                table_hbm.at[idx_v.at[pl.ds(off, CH)]], buf, sem
            ).wait()
            pltpu.sync_copy(buf, out_hbm.at[pl.ds(base + off, CH)])
            j2 = j + 2

            @pl.when(j2 < N_CH)
            def _():
                pltpu.async_copy(
                    table_hbm.at[idx_v.at[pl.ds(j2 * CH, CH)]], buf, sem
                )

        return carry

    lax.fori_loop(0, N_CH // 2, pair, 0, unroll=False)


def _sc_gather(idx, table):
    mesh = plsc.VectorSubcoreMesh(core_axis_name="c", subcore_axis_name="s")
    k = functools.partial(
        pl.kernel,
        mesh=mesh,
        out_type=jax.ShapeDtypeStruct((N, D), jnp.float32),
        scratch_types=[
            pltpu.VMEM((PER_W,), jnp.int32),
            pltpu.VMEM((CH, D), jnp.float32),
            pltpu.VMEM((CH, D), jnp.float32),
            pltpu.SemaphoreType.DMA,
            pltpu.SemaphoreType.DMA,
        ],
    )(_sc_gather_kernel)
    return k(idx, table)


BB = 64


def _mlp_kernel(g_ref, w1t_ref, b1_ref, w2t_ref, b2_ref, out_ref):
    h = jnp.maximum(
        jnp.dot(g_ref[...], w1t_ref[...], preferred_element_type=jnp.float32)
        + b1_ref[...],
        0.0,
    )
    out = jnp.dot(h, w2t_ref[...], preferred_element_type=jnp.float32) + b2_ref[...]
    out_ref[...] = out.reshape(BB, L, N_TAGS)


def _tc_mlp(g, w1t, b1, w2t, b2):
    grid = (B // BB,)
    return pl.pallas_call(
        _mlp_kernel,
        grid=grid,
        in_specs=[
            pl.BlockSpec((BB * L, D), lambda i: (i, 0)),
            pl.BlockSpec((D, D), lambda i: (0, 0)),
            pl.BlockSpec((1, D), lambda i: (0, 0)),
            pl.BlockSpec((D, N_TAGS), lambda i: (0, 0)),
            pl.BlockSpec((1, N_TAGS), lambda i: (0, 0)),
        ],
        out_specs=pl.BlockSpec((BB, L, N_TAGS), lambda i: (i, 0, 0)),
        out_shape=jax.ShapeDtypeStruct((B, L, N_TAGS), jnp.float32),
    )(g, w1t, b1, w2t, b2)


def kernel(x, emb, W1, b1, W2, b2):
    idx = x.reshape(-1).astype(jnp.int32)
    g = _sc_gather(idx, emb)
    return _tc_mlp(g, W1.T, b1.reshape(1, D), W2.T, b2.reshape(1, N_TAGS))


# speedup vs baseline: 23.4061x; 1.5293x over previous
import functools

import jax
import jax.numpy as jnp
from jax import lax
from jax.experimental import pallas as pl
from jax.experimental.pallas import tpu as pltpu
from jax.experimental.pallas import tpu_sc as plsc

B, L = 1024, 200
N = B * L
D = 128
N_TAGS = 45

_info = plsc.get_sparse_core_info()
NC, NS = _info.num_cores, _info.num_subcores
NW = NC * NS
PER_W = N // NW
CH = 128
N_CH = PER_W // CH


def _sc_gather_kernel(idx_hbm, table_hbm, out_hbm, idx_v, buf0, buf1, sem0, sem1):
    wid = lax.axis_index("s") * NC + lax.axis_index("c")
    base = wid * PER_W
    pltpu.sync_copy(idx_hbm.at[pl.ds(base, PER_W)], idx_v)

    pltpu.async_copy(table_hbm.at[idx_v.at[pl.ds(0, CH)]], buf0, sem0)
    pltpu.async_copy(table_hbm.at[idx_v.at[pl.ds(CH, CH)]], buf1, sem1)

    def pair(i, carry):
        for b, buf, sem in ((0, buf0, sem0), (1, buf1, sem1)):
            j = 2 * i + b
            off = j * CH
            pltpu.make_async_copy(
                table_hbm.at[idx_v.at[pl.ds(off, CH)]], buf, sem
            ).wait()
            pltpu.sync_copy(buf, out_hbm.at[pl.ds(base + off, CH)])
            j2 = j + 2

            @pl.when(j2 < N_CH)
            def _():
                pltpu.async_copy(
                    table_hbm.at[idx_v.at[pl.ds(j2 * CH, CH)]], buf, sem
                )

        return carry

    lax.fori_loop(0, N_CH // 2, pair, 0, unroll=False)


def _sc_gather(idx, table):
    mesh = plsc.VectorSubcoreMesh(core_axis_name="c", subcore_axis_name="s")
    k = functools.partial(
        pl.kernel,
        mesh=mesh,
        out_type=jax.ShapeDtypeStruct((N, D), jnp.float32),
        scratch_types=[
            pltpu.VMEM((PER_W,), jnp.int32),
            pltpu.VMEM((CH, D), jnp.float32),
            pltpu.VMEM((CH, D), jnp.float32),
            pltpu.SemaphoreType.DMA,
            pltpu.SemaphoreType.DMA,
        ],
    )(_sc_gather_kernel)
    return k(idx, table)


BL = 8


def _mlp_kernel(g_ref, w1t_ref, b1_ref, w2_ref, b2_ref, out_ref):
    for j in range(BL):
        h = jnp.maximum(
            jnp.dot(
                g_ref[pl.ds(j * B, B), :], w1t_ref[...],
                preferred_element_type=jnp.float32,
            )
            + b1_ref[...],
            0.0,
        )
        ot = (
            lax.dot_general(
                w2_ref[...], h, (((1,), (1,)), ((), ())),
                preferred_element_type=jnp.float32,
            )
            + b2_ref[...]
        )
        out_ref[:, j, :] = ot


def _tc_mlp_t(g, w1t, b1, w2, b2c):
    return pl.pallas_call(
        _mlp_kernel,
        grid=(L // BL,),
        in_specs=[
            pl.BlockSpec((BL * B, D), lambda i: (i, 0)),
            pl.BlockSpec((D, D), lambda i: (0, 0)),
            pl.BlockSpec((1, D), lambda i: (0, 0)),
            pl.BlockSpec((N_TAGS, D), lambda i: (0, 0)),
            pl.BlockSpec((N_TAGS, 1), lambda i: (0, 0)),
        ],
        out_specs=pl.BlockSpec((N_TAGS, BL, B), lambda i: (0, i, 0)),
        out_shape=jax.ShapeDtypeStruct((N_TAGS, L, B), jnp.float32),
    )(g, w1t, b1, w2, b2c)


def kernel(x, emb, W1, b1, W2, b2):
    idx = x.T.reshape(-1).astype(jnp.int32)
    g = _sc_gather(idx, emb)
    out_t = _tc_mlp_t(g, W1.T, b1.reshape(1, D), W2, b2.reshape(N_TAGS, 1))
    return lax.transpose(out_t, (2, 1, 0))
